# baseline (speedup 1.0000x reference)
import functools

import jax
import jax.numpy as jnp
from jax import lax
from jax.experimental import pallas as pl
from jax.experimental.pallas import tpu as pltpu
from jax.experimental.pallas import tpu_sc as plsc

_K0 = 0
_K1 = 42
_KS2 = _K0 ^ _K1 ^ 0x1BD11BDA

_ROTS = ((13, 15, 26, 6), (17, 29, 16, 24))
_KS = (_K0, _K1, _KS2)

_TINY = float(jnp.finfo(jnp.float32).tiny)
_NEG_INF = float("-inf")
_SQRT2 = 1.4142135381698608
_LN2 = 0.6931471805599453

_SC_S = 245760
_SC_CHUNK = 4096
_SC_UNROLL = 4


def _rotl(x, r):
    return (x << r) | lax.shift_right_logical(x, 32 - r)


def _threefry_bits(x1):
    x0 = x1
    x1 = _rotl(x1, 13)
    x1 = x0 ^ x1
    for r in _ROTS[0][1:]:
        x0 = x0 + x1
        x1 = _rotl(x1, r)
        x1 = x0 ^ x1
    x0 = x0 + jnp.int32(_KS[1])
    x1 = x1 + jnp.int32((_KS[2] + 1) & 0xFFFFFFFF)
    for i in range(1, 5):
        for r in _ROTS[i % 2]:
            x0 = x0 + x1
            x1 = _rotl(x1, r)
            x1 = x0 ^ x1
        x0 = x0 + jnp.int32(_KS[(i + 1) % 3])
        x1 = x1 + jnp.int32((_KS[(i + 2) % 3] + (i + 1)) & 0xFFFFFFFF)
    return x0 ^ x1


def _uniform_from_bits(bits):
    float_bits = lax.shift_right_logical(bits, 9) | jnp.int32(0x3F800000)
    f = lax.bitcast_convert_type(float_bits, jnp.float32) - jnp.float32(1.0)
    return jnp.maximum(jnp.float32(_TINY), f)


def _gumbel_from_bits(bits):
    u = _uniform_from_bits(bits)
    return -jnp.log(-jnp.log(u))


def _neg_log_u(bits):
    u = _uniform_from_bits(bits)
    iu = lax.bitcast_convert_type(u, jnp.int32)
    e = lax.shift_right_arithmetic(iu, 23) - jnp.int32(127)
    m = lax.bitcast_convert_type((iu & jnp.int32(0x007FFFFF))
                                 | jnp.int32(0x3F800000), jnp.float32)
    big = m > jnp.float32(_SQRT2)
    mh = jnp.where(big, jnp.float32(0.5) * m, m)
    eh = jnp.where(big, e + jnp.int32(1), e).astype(jnp.float32)
    t = mh - jnp.float32(1.0)
    s = t / (mh + jnp.float32(1.0))
    s2 = s * s
    p = jnp.float32(2.0 / 9.0)
    p = p * s2 + jnp.float32(2.0 / 7.0)
    p = p * s2 + jnp.float32(2.0 / 5.0)
    p = p * s2 + jnp.float32(2.0 / 3.0)
    p = p * s2 + jnp.float32(2.0)
    logm = p * s
    return -(eh * jnp.float32(_LN2) + logm)


def _tc_kernel(logits_ref, val_ref, col_ref, *, vocab, limit, block_w,
               chunk_w, nblocks):
    j = pl.program_id(0)

    @pl.when(j == 0)
    def _init():
        val_ref[...] = jnp.full_like(val_ref, jnp.float32(_NEG_INF))
        col_ref[...] = jnp.zeros_like(col_ref)

    rows, _ = logits_ref.shape
    base = j * block_w
    row_iota = lax.broadcasted_iota(jnp.int32, (rows, chunk_w), 0)
    lane_iota = lax.broadcasted_iota(jnp.int32, (rows, chunk_w), 1)
    rowlane = row_iota * jnp.int32(vocab) + lane_iota + jnp.int32(_K1)

    def run_chunks(masked):
        val = val_ref[...]
        col = col_ref[...]
        for t in range(block_w // chunk_w):
            cbase = base + t * chunk_w
            g = _gumbel_from_bits(_threefry_bits(rowlane + cbase))
            score = logits_ref[:, t * chunk_w:(t + 1) * chunk_w] + g
            c = cbase + lane_iota
            if masked:
                score = jnp.where(c < limit, score, jnp.float32(_NEG_INF))
            better = score > val
            val = jnp.where(better, score, val)
            col = jnp.where(better, c, col)
        val_ref[...] = val
        col_ref[...] = col

    @pl.when(j < nblocks - 1)
    def _main():
        run_chunks(masked=False)

    @pl.when(j == nblocks - 1)
    def _last():
        run_chunks(masked=True)


def _tc_sample(logits, limit, block_w=4096, chunk_w=256):
    rows, vocab = logits.shape
    nblocks = pl.cdiv(limit, block_w)
    return pl.pallas_call(
        functools.partial(_tc_kernel, vocab=vocab, limit=limit,
                          block_w=block_w, chunk_w=chunk_w, nblocks=nblocks),
        grid=(nblocks,),
        in_specs=[pl.BlockSpec((rows, block_w), lambda j: (0, j))],
        out_specs=[pl.BlockSpec((rows, chunk_w), lambda j: (0, 0)),
                   pl.BlockSpec((rows, chunk_w), lambda j: (0, 0))],
        out_shape=[jax.ShapeDtypeStruct((rows, chunk_w), jnp.float32),
                   jax.ShapeDtypeStruct((rows, chunk_w), jnp.int32)],
        compiler_params=pltpu.CompilerParams(
            dimension_semantics=("arbitrary",)),
    )(logits)


def _sc_sample(shard_logits, rows, vocab):
    col0 = vocab - _SC_S
    nchunk = _SC_S // _SC_CHUNK
    nvec = _SC_CHUNK // 16
    mesh = plsc.VectorSubcoreMesh(core_axis_name="c", subcore_axis_name="s")

    @functools.partial(
        pl.kernel, mesh=mesh,
        out_type=[jax.ShapeDtypeStruct((rows, 16), jnp.float32),
                  jax.ShapeDtypeStruct((rows, 16), jnp.int32)],
        scratch_types=[pltpu.VMEM((_SC_CHUNK,), jnp.float32),
                       pltpu.VMEM((_SC_CHUNK,), jnp.float32),
                       pltpu.VMEM((16,), jnp.float32),
                       pltpu.VMEM((16,), jnp.int32),
                       pltpu.SemaphoreType.DMA,
                       pltpu.SemaphoreType.DMA],
    )
    def sc_body(flat_hbm, out_val, out_col, buf0, buf1, vstash, cstash,
                sem0, sem1):
        wid = lax.axis_index("s") * 2 + lax.axis_index("c")
        rowbase = wid * jnp.int32(vocab)
        start0 = wid * jnp.int32(_SC_S)
        bufs = (buf0, buf1)
        sems = (sem0, sem1)
        pltpu.make_async_copy(
            flat_hbm.at[pl.ds(start0, _SC_CHUNK)], buf0, sem0).start()
        pltpu.make_async_copy(
            flat_hbm.at[pl.ds(start0 + _SC_CHUNK, _SC_CHUNK)], buf1,
            sem1).start()
        iota16 = lax.iota(jnp.int32, 16)
        rowk = rowbase + jnp.int32(_K1)

        def pair_body(pair, carry):
            minv, minc = carry
            for b in range(2):
                kk = 2 * pair + b
                buf = bufs[b]
                sem = sems[b]
                pltpu.make_async_copy(
                    flat_hbm.at[pl.ds(start0, _SC_CHUNK)], buf, sem).wait()
                cb = jnp.int32(col0) + kk * jnp.int32(_SC_CHUNK)

                def inner(v, carry2):
                    minv, minc = carry2
                    for uu in range(_SC_UNROLL):
                        off = v * (16 * _SC_UNROLL) + uu * 16
                        col = cb + off + iota16
                        E = _neg_log_u(_threefry_bits(col + rowk))
                        lvec = buf[pl.ds(off, 16)]
                        sc = E * jnp.exp(-lvec)
                        better = sc < minv
                        minv = jnp.where(better, sc, minv)
                        minc = jnp.where(better, col, minc)
                    return minv, minc

                minv, minc = lax.fori_loop(0, nvec // _SC_UNROLL, inner,
                                           (minv, minc))

                @pl.when(kk + 2 < nchunk)
                def _prefetch():
                    pltpu.make_async_copy(
                        flat_hbm.at[pl.ds(start0 + (kk + 2) * _SC_CHUNK,
                                          _SC_CHUNK)], buf, sem).start()
            return minv, minc

        minv0 = jnp.full((16,), jnp.float32(jnp.inf))
        minc0 = jnp.zeros((16,), jnp.int32)
        minv, minc = lax.fori_loop(0, nchunk // 2, pair_body, (minv0, minc0))
        vstash[...] = minv
        cstash[...] = minc
        pltpu.sync_copy(vstash, out_val.at[wid])
        pltpu.sync_copy(cstash, out_col.at[wid])

    return sc_body(shard_logits)


@jax.jit
def _sample(logits):
    rows, vocab = logits.shape
    limit = vocab - _SC_S
    tc_val, tc_col = _tc_sample(logits, limit=limit)
    sc_val, sc_col = _sc_sample(logits[:, limit:].reshape(-1), rows, vocab)
    sc_gl = -jnp.log(sc_val)
    vals = jnp.concatenate([tc_val, sc_gl], axis=1)
    cols = jnp.concatenate([tc_col, sc_col], axis=1)
    vmax = jnp.max(vals, axis=1, keepdims=True)
    best = jnp.min(jnp.where(vals == vmax, cols, jnp.int32(2**31 - 1)),
                   axis=1, keepdims=True)
    return best


def kernel(logits):
    return _sample(logits).astype(jnp.int64)


# speedup vs baseline: 1.0571x; 1.0571x over previous
import functools

import jax
import jax.numpy as jnp
from jax import lax
from jax.experimental import pallas as pl
from jax.experimental.pallas import tpu as pltpu
from jax.experimental.pallas import tpu_sc as plsc

_K0 = 0
_K1 = 42
_KS2 = _K0 ^ _K1 ^ 0x1BD11BDA

_ROTS = ((13, 15, 26, 6), (17, 29, 16, 24))
_KS = (_K0, _K1, _KS2)

_TINY = float(jnp.finfo(jnp.float32).tiny)
_NEG_INF = float("-inf")
_SQRT2 = 1.4142135381698608
_LN2 = 0.6931471805599453

_SC_S = 221184
_SC_CHUNK = 4096
_SC_UNROLL = 4


def _rotl(x, r):
    return (x << r) | lax.shift_right_logical(x, 32 - r)


def _threefry_bits(x1):
    x0 = x1
    x1 = _rotl(x1, 13)
    x1 = x0 ^ x1
    for r in _ROTS[0][1:]:
        x0 = x0 + x1
        x1 = _rotl(x1, r)
        x1 = x0 ^ x1
    x0 = x0 + jnp.int32(_KS[1])
    x1 = x1 + jnp.int32((_KS[2] + 1) & 0xFFFFFFFF)
    for i in range(1, 5):
        for r in _ROTS[i % 2]:
            x0 = x0 + x1
            x1 = _rotl(x1, r)
            x1 = x0 ^ x1
        x0 = x0 + jnp.int32(_KS[(i + 1) % 3])
        x1 = x1 + jnp.int32((_KS[(i + 2) % 3] + (i + 1)) & 0xFFFFFFFF)
    return x0 ^ x1


def _uniform_from_bits(bits):
    float_bits = lax.shift_right_logical(bits, 9) | jnp.int32(0x3F800000)
    f = lax.bitcast_convert_type(float_bits, jnp.float32) - jnp.float32(1.0)
    return jnp.maximum(jnp.float32(_TINY), f)


def _gumbel_from_bits(bits):
    u = _uniform_from_bits(bits)
    return -jnp.log(-jnp.log(u))


def _neg_log_u(bits):
    u = _uniform_from_bits(bits)
    iu = lax.bitcast_convert_type(u, jnp.int32)
    e = lax.shift_right_arithmetic(iu, 23) - jnp.int32(127)
    m = lax.bitcast_convert_type((iu & jnp.int32(0x007FFFFF))
                                 | jnp.int32(0x3F800000), jnp.float32)
    big = m > jnp.float32(_SQRT2)
    mh = jnp.where(big, jnp.float32(0.5) * m, m)
    eh = jnp.where(big, e + jnp.int32(1), e).astype(jnp.float32)
    t = mh - jnp.float32(1.0)
    s = t / (mh + jnp.float32(1.0))
    s2 = s * s
    p = jnp.float32(2.0 / 9.0)
    p = p * s2 + jnp.float32(2.0 / 7.0)
    p = p * s2 + jnp.float32(2.0 / 5.0)
    p = p * s2 + jnp.float32(2.0 / 3.0)
    p = p * s2 + jnp.float32(2.0)
    logm = p * s
    return -(eh * jnp.float32(_LN2) + logm)


def _tc_kernel(logits_ref, val_ref, col_ref, *, vocab, limit, block_w,
               chunk_w, nblocks):
    j = pl.program_id(0)

    @pl.when(j == 0)
    def _init():
        val_ref[...] = jnp.full_like(val_ref, jnp.float32(_NEG_INF))
        col_ref[...] = jnp.zeros_like(col_ref)

    rows, _ = logits_ref.shape
    base = j * block_w
    row_iota = lax.broadcasted_iota(jnp.int32, (rows, chunk_w), 0)
    lane_iota = lax.broadcasted_iota(jnp.int32, (rows, chunk_w), 1)
    rowlane = row_iota * jnp.int32(vocab) + lane_iota + jnp.int32(_K1)

    def run_chunks(masked):
        val = val_ref[...]
        col = col_ref[...]
        for t in range(block_w // chunk_w):
            cbase = base + t * chunk_w
            g = _gumbel_from_bits(_threefry_bits(rowlane + cbase))
            score = logits_ref[:, t * chunk_w:(t + 1) * chunk_w] + g
            c = cbase + lane_iota
            if masked:
                score = jnp.where(c < limit, score, jnp.float32(_NEG_INF))
            better = score > val
            val = jnp.where(better, score, val)
            col = jnp.where(better, c, col)
        val_ref[...] = val
        col_ref[...] = col

    @pl.when(j < nblocks - 1)
    def _main():
        run_chunks(masked=False)

    @pl.when(j == nblocks - 1)
    def _last():
        run_chunks(masked=True)


def _tc_sample(logits, limit, block_w=4096, chunk_w=256):
    rows, vocab = logits.shape
    nblocks = pl.cdiv(limit, block_w)
    return pl.pallas_call(
        functools.partial(_tc_kernel, vocab=vocab, limit=limit,
                          block_w=block_w, chunk_w=chunk_w, nblocks=nblocks),
        grid=(nblocks,),
        in_specs=[pl.BlockSpec((rows, block_w), lambda j: (0, j))],
        out_specs=[pl.BlockSpec((rows, chunk_w), lambda j: (0, 0)),
                   pl.BlockSpec((rows, chunk_w), lambda j: (0, 0))],
        out_shape=[jax.ShapeDtypeStruct((rows, chunk_w), jnp.float32),
                   jax.ShapeDtypeStruct((rows, chunk_w), jnp.int32)],
        compiler_params=pltpu.CompilerParams(
            dimension_semantics=("arbitrary",)),
    )(logits)


def _sc_sample(shard_logits, rows, vocab):
    col0 = vocab - _SC_S
    nchunk = _SC_S // _SC_CHUNK
    nvec = _SC_CHUNK // 16
    mesh = plsc.VectorSubcoreMesh(core_axis_name="c", subcore_axis_name="s")

    @functools.partial(
        pl.kernel, mesh=mesh,
        out_type=[jax.ShapeDtypeStruct((rows, 16), jnp.float32),
                  jax.ShapeDtypeStruct((rows, 16), jnp.int32)],
        scratch_types=[pltpu.VMEM((_SC_CHUNK,), jnp.float32),
                       pltpu.VMEM((_SC_CHUNK,), jnp.float32),
                       pltpu.VMEM((16,), jnp.float32),
                       pltpu.VMEM((16,), jnp.int32),
                       pltpu.SemaphoreType.DMA,
                       pltpu.SemaphoreType.DMA],
    )
    def sc_body(flat_hbm, out_val, out_col, buf0, buf1, vstash, cstash,
                sem0, sem1):
        wid = lax.axis_index("s") * 2 + lax.axis_index("c")
        rowbase = wid * jnp.int32(vocab)
        start0 = wid * jnp.int32(_SC_S)
        bufs = (buf0, buf1)
        sems = (sem0, sem1)
        pltpu.make_async_copy(
            flat_hbm.at[pl.ds(start0, _SC_CHUNK)], buf0, sem0).start()
        pltpu.make_async_copy(
            flat_hbm.at[pl.ds(start0 + _SC_CHUNK, _SC_CHUNK)], buf1,
            sem1).start()
        iota16 = lax.iota(jnp.int32, 16)
        rowk = rowbase + jnp.int32(_K1)

        def pair_body(pair, carry):
            minv, minc = carry
            for b in range(2):
                kk = 2 * pair + b
                buf = bufs[b]
                sem = sems[b]
                pltpu.make_async_copy(
                    flat_hbm.at[pl.ds(start0, _SC_CHUNK)], buf, sem).wait()
                cb = jnp.int32(col0) + kk * jnp.int32(_SC_CHUNK)

                @plsc.parallel_loop(0, nvec, step=1, unroll=_SC_UNROLL,
                                    carry=(minv, minc))
                def inner(v, carry2):
                    minv, minc = carry2
                    off = v * 16
                    col = cb + off + iota16
                    E = _neg_log_u(_threefry_bits(col + rowk))
                    lvec = buf[pl.ds(off, 16)]
                    sc = E * jnp.exp(-lvec)
                    better = sc < minv
                    return (jnp.where(better, sc, minv),
                            jnp.where(better, col, minc))

                minv, minc = inner

                @pl.when(kk + 2 < nchunk)
                def _prefetch():
                    pltpu.make_async_copy(
                        flat_hbm.at[pl.ds(start0 + (kk + 2) * _SC_CHUNK,
                                          _SC_CHUNK)], buf, sem).start()
            return minv, minc

        minv0 = jnp.full((16,), jnp.float32(jnp.inf))
        minc0 = jnp.zeros((16,), jnp.int32)
        minv, minc = lax.fori_loop(0, nchunk // 2, pair_body, (minv0, minc0))
        vstash[...] = minv
        cstash[...] = minc
        pltpu.sync_copy(vstash, out_val.at[wid])
        pltpu.sync_copy(cstash, out_col.at[wid])

    return sc_body(shard_logits)


@jax.jit
def _sample(logits):
    rows, vocab = logits.shape
    limit = vocab - _SC_S
    tc_val, tc_col = _tc_sample(logits, limit=limit)
    sc_val, sc_col = _sc_sample(logits[:, limit:].reshape(-1), rows, vocab)
    sc_gl = -jnp.log(sc_val)
    vals = jnp.concatenate([tc_val, sc_gl], axis=1)
    cols = jnp.concatenate([tc_col, sc_col], axis=1)
    vmax = jnp.max(vals, axis=1, keepdims=True)
    best = jnp.min(jnp.where(vals == vmax, cols, jnp.int32(2**31 - 1)),
                   axis=1, keepdims=True)
    return best


def kernel(logits):
    return _sample(logits).astype(jnp.int64)
